# baseline (speedup 1.0000x reference)
import functools

import jax
import jax.numpy as jnp
from jax import lax
from jax.experimental import pallas as pl
from jax.experimental.pallas import tpu as pltpu
from jax.experimental.pallas import tpu_sc as plsc

_VOCAB = 1_000_000
_EMBED = 64
_BATCH = 4096
_SEQ = 200
_HPAD = 128
_GW = 64
_OW = 16

_ROW_BLK = 20000
_NC = 2
_NS = 16
_NW = _NC * _NS
_EPW = _BATCH // _NW
_HSEQ = _SEQ // 2


def _g_body(tbl_ref, w1_ref, b1_ref, w23_ref, out_ref):
    h = jnp.tanh(
        jnp.dot(
            tbl_ref[...].astype(jnp.bfloat16),
            w1_ref[...],
            preferred_element_type=jnp.float32,
        )
        + b1_ref[...]
    )
    out_ref[...] = jnp.dot(
        h.astype(jnp.bfloat16), w23_ref[...], preferred_element_type=jnp.float32
    )


def _precompute_g(table, w1p, b1p, w23p):
    return pl.pallas_call(
        _g_body,
        grid=(_VOCAB // _ROW_BLK,),
        in_specs=[
            pl.BlockSpec((_ROW_BLK, _EMBED), lambda i: (i, 0)),
            pl.BlockSpec((_EMBED, _HPAD), lambda i: (0, 0)),
            pl.BlockSpec((1, _HPAD), lambda i: (0, 0)),
            pl.BlockSpec((_HPAD, _GW), lambda i: (0, 0)),
        ],
        out_specs=pl.BlockSpec((_ROW_BLK, _GW), lambda i: (i, 0)),
        out_shape=jax.ShapeDtypeStruct((_VOCAB, _GW), jnp.float32),
    )(table, w1p, b1p, w23p)


def _reduce_elem(rows_ref, acc_ref, e):
    def red(j, accs):
        a0, a1, a2, a3 = accs
        r = j * 8
        a0 = a0 + rows_ref[r, 0:_OW] + rows_ref[r + 4, 0:_OW]
        a1 = a1 + rows_ref[r + 1, 0:_OW] + rows_ref[r + 5, 0:_OW]
        a2 = a2 + rows_ref[r + 2, 0:_OW] + rows_ref[r + 6, 0:_OW]
        a3 = a3 + rows_ref[r + 3, 0:_OW] + rows_ref[r + 7, 0:_OW]
        return (a0, a1, a2, a3)

    z = jnp.zeros((_OW,), jnp.float32)
    a0, a1, a2, a3 = lax.fori_loop(0, _SEQ // 8, red, (z, z, z, z))
    acc_ref[e] = (a0 + a1) + (a2 + a3)


def _sc_body(idx_hbm, g_hbm, out_hbm, idx_v, rows0, rows1, acc_v, sem0, sem1):
    c = lax.axis_index("c")
    s = lax.axis_index("s")
    wid = s * _NC + c
    base = wid * _EPW

    pltpu.sync_copy(idx_hbm.at[pl.ds(2 * base, 2 * _EPW)], idx_v)

    def fire(e, rows, sem):
        pltpu.make_async_copy(
            g_hbm.at[idx_v.at[2 * e]], rows.at[pl.ds(0, _HSEQ)], sem
        ).start()
        pltpu.make_async_copy(
            g_hbm.at[idx_v.at[2 * e + 1]], rows.at[pl.ds(_HSEQ, _HSEQ)], sem
        ).start()

    def drain(e, rows, sem):
        pltpu.make_async_copy(
            g_hbm.at[idx_v.at[2 * e]], rows.at[pl.ds(0, _HSEQ)], sem
        ).wait()
        pltpu.make_async_copy(
            g_hbm.at[idx_v.at[2 * e + 1]], rows.at[pl.ds(_HSEQ, _HSEQ)], sem
        ).wait()

    fire(0, rows0, sem0)

    def body(i, carry):
        e0 = 2 * i
        e1 = 2 * i + 1
        fire(e1, rows1, sem1)
        drain(e0, rows0, sem0)
        _reduce_elem(rows0, acc_v, e0)

        @pl.when(e0 + 2 < _EPW)
        def _():
            fire(e0 + 2, rows0, sem0)

        drain(e1, rows1, sem1)
        _reduce_elem(rows1, acc_v, e1)
        return carry

    lax.fori_loop(0, _EPW // 2, body, 0)
    pltpu.sync_copy(acc_v, out_hbm.at[pl.ds(base, _EPW)])


def _sc_gather_sum(idx2, g):
    mesh = plsc.VectorSubcoreMesh(core_axis_name="c", subcore_axis_name="s")
    return pl.kernel(
        _sc_body,
        out_type=jax.ShapeDtypeStruct((_BATCH, _OW), jnp.float32),
        mesh=mesh,
        scratch_types=[
            pltpu.VMEM((2 * _EPW, _HSEQ), jnp.int32),
            pltpu.VMEM((_SEQ, _GW), jnp.float32),
            pltpu.VMEM((_SEQ, _GW), jnp.float32),
            pltpu.VMEM((_EPW, _OW), jnp.float32),
            pltpu.SemaphoreType.DMA,
            pltpu.SemaphoreType.DMA,
        ],
        compiler_params=pltpu.CompilerParams(use_tc_tiling_on_sc=False),
    )(idx2, g)


def kernel(inputs, table, W1, b1, W2, b2, W3, b3):
    idx2 = inputs.astype(jnp.int32).reshape(_BATCH * 2, _HSEQ)
    w23 = W3 @ W2
    w1p = (
        jnp.zeros((_EMBED, _HPAD), jnp.float32)
        .at[:, :100]
        .set(W1.T)
        .astype(jnp.bfloat16)
    )
    b1p = jnp.zeros((1, _HPAD), jnp.float32).at[0, :100].set(b1)
    w23p = (
        jnp.zeros((_HPAD, _GW), jnp.float32)
        .at[:100, :5]
        .set(w23.T)
        .astype(jnp.bfloat16)
    )
    g = _precompute_g(table, w1p, b1p, w23p)
    s16 = _sc_gather_sum(idx2, g)
    const = _SEQ * (b2 @ W3.T + b3)
    return s16[:, :5] + const


# speedup vs baseline: 1.1465x; 1.1465x over previous
import functools

import jax
import jax.numpy as jnp
from jax import lax
from jax.experimental import pallas as pl
from jax.experimental.pallas import tpu as pltpu
from jax.experimental.pallas import tpu_sc as plsc

_VOCAB = 1_000_000
_EMBED = 64
_BATCH = 4096
_SEQ = 200
_HPAD = 128
_GW = 16

_ROW_BLK = 20000
_NC = 2
_NS = 16
_NW = _NC * _NS
_EPW = _BATCH // _NW
_HSEQ = _SEQ // 2
_NBUF = 8


def _g_body(tbl_ref, w1_ref, b1_ref, w23_ref, out_ref):
    h = jnp.tanh(
        jnp.dot(
            tbl_ref[...].astype(jnp.bfloat16),
            w1_ref[...],
            preferred_element_type=jnp.float32,
        )
        + b1_ref[...]
    )
    out_ref[...] = jnp.dot(
        h.astype(jnp.bfloat16), w23_ref[...], preferred_element_type=jnp.float32
    )


def _precompute_g(table, w1p, b1p, w23p):
    return pl.pallas_call(
        _g_body,
        grid=(_VOCAB // _ROW_BLK,),
        in_specs=[
            pl.BlockSpec((_ROW_BLK, _EMBED), lambda i: (i, 0)),
            pl.BlockSpec((_EMBED, _HPAD), lambda i: (0, 0)),
            pl.BlockSpec((1, _HPAD), lambda i: (0, 0)),
            pl.BlockSpec((_HPAD, _GW), lambda i: (0, 0)),
        ],
        out_specs=pl.BlockSpec((_ROW_BLK, _GW), lambda i: (i, 0)),
        out_shape=jax.ShapeDtypeStruct((_VOCAB, _GW), jnp.float32),
    )(table, w1p, b1p, w23p)


def _reduce_elem(rows_ref, acc_ref, e):
    def red(j, accs):
        a0, a1, a2, a3 = accs
        r = j * 8
        a0 = a0 + rows_ref[r] + rows_ref[r + 4]
        a1 = a1 + rows_ref[r + 1] + rows_ref[r + 5]
        a2 = a2 + rows_ref[r + 2] + rows_ref[r + 6]
        a3 = a3 + rows_ref[r + 3] + rows_ref[r + 7]
        return (a0, a1, a2, a3)

    z = jnp.zeros((_GW,), jnp.float32)
    a0, a1, a2, a3 = lax.fori_loop(0, _SEQ // 8, red, (z, z, z, z))
    acc_ref[e] = (a0 + a1) + (a2 + a3)


def _sc_body(idx_hbm, g_hbm, out_hbm, idx_v, *rest):
    rows = rest[:_NBUF]
    acc_v = rest[_NBUF]
    sems = rest[_NBUF + 1 : _NBUF + 1 + _NBUF]
    c = lax.axis_index("c")
    s = lax.axis_index("s")
    wid = s * _NC + c
    base = wid * _EPW

    pltpu.sync_copy(idx_hbm.at[pl.ds(2 * base, 2 * _EPW)], idx_v)

    def fire(e, buf, sem):
        pltpu.make_async_copy(
            g_hbm.at[idx_v.at[2 * e]], buf.at[pl.ds(0, _HSEQ)], sem
        ).start()
        pltpu.make_async_copy(
            g_hbm.at[idx_v.at[2 * e + 1]], buf.at[pl.ds(_HSEQ, _HSEQ)], sem
        ).start()

    def drain(e, buf, sem):
        pltpu.make_async_copy(
            g_hbm.at[idx_v.at[2 * e]], buf.at[pl.ds(0, _HSEQ)], sem
        ).wait()
        pltpu.make_async_copy(
            g_hbm.at[idx_v.at[2 * e + 1]], buf.at[pl.ds(_HSEQ, _HSEQ)], sem
        ).wait()

    for b in range(_NBUF):
        fire(b, rows[b], sems[b])

    def body(i, carry):
        e0 = _NBUF * i
        for b in range(_NBUF):
            e = e0 + b
            drain(e, rows[b], sems[b])
            _reduce_elem(rows[b], acc_v, e)

            @pl.when(e + _NBUF < _EPW)
            def _():
                fire(e + _NBUF, rows[b], sems[b])

        return carry

    lax.fori_loop(0, _EPW // _NBUF, body, 0)
    pltpu.sync_copy(acc_v, out_hbm.at[pl.ds(base, _EPW)])


def _sc_gather_sum(idx2, g):
    mesh = plsc.VectorSubcoreMesh(core_axis_name="c", subcore_axis_name="s")
    return pl.kernel(
        _sc_body,
        out_type=jax.ShapeDtypeStruct((_BATCH, _GW), jnp.float32),
        mesh=mesh,
        scratch_types=(
            [pltpu.VMEM((2 * _EPW, _HSEQ), jnp.int32)]
            + [pltpu.VMEM((_SEQ, _GW), jnp.float32) for _ in range(_NBUF)]
            + [pltpu.VMEM((_EPW, _GW), jnp.float32)]
            + [pltpu.SemaphoreType.DMA for _ in range(_NBUF)]
        ),
        compiler_params=pltpu.CompilerParams(use_tc_tiling_on_sc=False),
    )(idx2, g)


def kernel(inputs, table, W1, b1, W2, b2, W3, b3):
    idx2 = inputs.astype(jnp.int32).reshape(_BATCH * 2, _HSEQ)
    w23 = W3 @ W2
    w1p = (
        jnp.zeros((_EMBED, _HPAD), jnp.float32)
        .at[:, :100]
        .set(W1.T)
        .astype(jnp.bfloat16)
    )
    b1p = jnp.zeros((1, _HPAD), jnp.float32).at[0, :100].set(b1)
    w23p = (
        jnp.zeros((_HPAD, _GW), jnp.float32)
        .at[:100, :5]
        .set(w23.T)
        .astype(jnp.bfloat16)
    )
    g = _precompute_g(table, w1p, b1p, w23p)
    s16 = _sc_gather_sum(idx2, g)
    const = _SEQ * (b2 @ W3.T + b3)
    return s16[:, :5] + const


# speedup vs baseline: 1.1513x; 1.0042x over previous
import functools

import jax
import jax.numpy as jnp
from jax import lax
from jax.experimental import pallas as pl
from jax.experimental.pallas import tpu as pltpu
from jax.experimental.pallas import tpu_sc as plsc

_VOCAB = 1_000_000
_EMBED = 64
_BATCH = 4096
_SEQ = 200
_HPAD = 128
_GW = 16

_ROW_BLK = 20000
_NC = 2
_NS = 16
_NW = _NC * _NS
_EPW = _BATCH // _NW
_HSEQ = _SEQ // 2
_NBUF = 8


def _g_body(tbl_ref, w1_ref, b1_ref, w23_ref, out_ref):
    h = jnp.tanh(
        jnp.dot(
            tbl_ref[...].astype(jnp.bfloat16),
            w1_ref[...],
            preferred_element_type=jnp.float32,
        )
        + b1_ref[...]
    )
    out_ref[...] = jnp.dot(
        h.astype(jnp.bfloat16), w23_ref[...], preferred_element_type=jnp.float32
    )


def _precompute_g(table, w1p, b1p, w23p):
    return pl.pallas_call(
        _g_body,
        grid=(_VOCAB // _ROW_BLK,),
        in_specs=[
            pl.BlockSpec((_ROW_BLK, _EMBED), lambda i: (i, 0)),
            pl.BlockSpec((_EMBED, _HPAD), lambda i: (0, 0)),
            pl.BlockSpec((1, _HPAD), lambda i: (0, 0)),
            pl.BlockSpec((_HPAD, _GW), lambda i: (0, 0)),
        ],
        out_specs=pl.BlockSpec((_ROW_BLK, _GW), lambda i: (i, 0)),
        out_shape=jax.ShapeDtypeStruct((_VOCAB, _GW), jnp.float32),
    )(table, w1p, b1p, w23p)


def _reduce_elem(rows_ref, acc_ref, e, off=0):
    def red(j, accs):
        a0, a1, a2, a3 = accs
        r = off + j * 8
        a0 = a0 + rows_ref[r] + rows_ref[r + 4]
        a1 = a1 + rows_ref[r + 1] + rows_ref[r + 5]
        a2 = a2 + rows_ref[r + 2] + rows_ref[r + 6]
        a3 = a3 + rows_ref[r + 3] + rows_ref[r + 7]
        return (a0, a1, a2, a3)

    z = jnp.zeros((_GW,), jnp.float32)
    a0, a1, a2, a3 = lax.fori_loop(0, _SEQ // 8, red, (z, z, z, z))
    acc_ref[e] = (a0 + a1) + (a2 + a3)


_CHK = 4
_CROWS = _CHK * _SEQ
_NCHK = _EPW // _CHK


def _sc_body(idx_hbm, g_hbm, out_hbm, idx_v, *rest):
    rows = rest[:_NBUF]
    acc_v = rest[_NBUF]
    sems = rest[_NBUF + 1 : _NBUF + 1 + _NBUF]
    c = lax.axis_index("c")
    s = lax.axis_index("s")
    wid = s * _NC + c
    base = wid * _EPW

    pltpu.sync_copy(idx_hbm.at[pl.ds(base * _SEQ, _EPW * _SEQ)], idx_v)

    def fire(ch, buf, sem):
        pltpu.make_async_copy(
            g_hbm.at[idx_v.at[pl.ds(ch * _CROWS, _CROWS)]], buf, sem
        ).start()

    def drain(ch, buf, sem):
        pltpu.make_async_copy(
            g_hbm.at[idx_v.at[pl.ds(ch * _CROWS, _CROWS)]], buf, sem
        ).wait()

    for b in range(_NBUF):
        fire(b, rows[b], sems[b])

    def body(i, carry):
        c0 = _NBUF * i
        for b in range(_NBUF):
            ch = c0 + b
            drain(ch, rows[b], sems[b])
            for k in range(_CHK):
                _reduce_elem(rows[b], acc_v, ch * _CHK + k, off=k * _SEQ)

            @pl.when(ch + _NBUF < _NCHK)
            def _():
                fire(ch + _NBUF, rows[b], sems[b])

        return carry

    lax.fori_loop(0, _NCHK // _NBUF, body, 0)
    pltpu.sync_copy(acc_v, out_hbm.at[pl.ds(base, _EPW)])


def _sc_gather_sum(idx1, g):
    mesh = plsc.VectorSubcoreMesh(core_axis_name="c", subcore_axis_name="s")
    return pl.kernel(
        _sc_body,
        out_type=jax.ShapeDtypeStruct((_BATCH, _GW), jnp.float32),
        mesh=mesh,
        scratch_types=(
            [pltpu.VMEM((_EPW * _SEQ,), jnp.int32)]
            + [pltpu.VMEM((_CROWS, _GW), jnp.float32) for _ in range(_NBUF)]
            + [pltpu.VMEM((_EPW, _GW), jnp.float32)]
            + [pltpu.SemaphoreType.DMA for _ in range(_NBUF)]
        ),
        compiler_params=pltpu.CompilerParams(use_tc_tiling_on_sc=False),
    )(idx1, g)


def kernel(inputs, table, W1, b1, W2, b2, W3, b3):
    idx1 = inputs.astype(jnp.int32).reshape(_BATCH * _SEQ)
    w23 = W3 @ W2
    w1p = (
        jnp.zeros((_EMBED, _HPAD), jnp.float32)
        .at[:, :100]
        .set(W1.T)
        .astype(jnp.bfloat16)
    )
    b1p = jnp.zeros((1, _HPAD), jnp.float32).at[0, :100].set(b1)
    w23p = (
        jnp.zeros((_HPAD, _GW), jnp.float32)
        .at[:100, :5]
        .set(w23.T)
        .astype(jnp.bfloat16)
    )
    g = _precompute_g(table, w1p, b1p, w23p)
    s16 = _sc_gather_sum(idx1, g)
    const = _SEQ * (b2 @ W3.T + b3)
    return s16[:, :5] + const


# speedup vs baseline: 1.1519x; 1.0005x over previous
import functools

import jax
import jax.numpy as jnp
from jax import lax
from jax.experimental import pallas as pl
from jax.experimental.pallas import tpu as pltpu
from jax.experimental.pallas import tpu_sc as plsc

_VOCAB = 1_000_000
_EMBED = 64
_BATCH = 4096
_SEQ = 200
_HPAD = 128
_GW = 16

_ROW_BLK = 20000
_NC = 2
_NS = 16
_NW = _NC * _NS
_EPW = _BATCH // _NW
_HSEQ = _SEQ // 2
_NBUF = 8


def _g_body(tbl_ref, w1_ref, b1_ref, w23_ref, out_ref):
    h = jnp.tanh(
        jnp.dot(
            tbl_ref[...].astype(jnp.bfloat16),
            w1_ref[...],
            preferred_element_type=jnp.float32,
        )
        + b1_ref[...]
    )
    out_ref[...] = jnp.dot(
        h.astype(jnp.bfloat16), w23_ref[...], preferred_element_type=jnp.float32
    )


_CH = 10000
_NCHUNK = _VOCAB // _CH
_NB = 5
_NSTEP = _NCHUNK // _NB


def _g_manual_body(tbl_hbm, w1_ref, b1_ref, w23_ref, out_hbm, *rest):
    ibufs = rest[:_NB]
    obufs = rest[_NB : 2 * _NB]
    isems = rest[2 * _NB : 3 * _NB]
    osems = rest[3 * _NB : 4 * _NB]
    i = pl.program_id(0)

    def in_copy(ch, b):
        return pltpu.make_async_copy(
            tbl_hbm.at[pl.ds(ch * _CH, _CH), :], ibufs[b], isems[b]
        )

    def out_copy(ch, b):
        return pltpu.make_async_copy(
            obufs[b], out_hbm.at[pl.ds(ch * _CH, _CH), :], osems[b]
        )

    @pl.when(i == 0)
    def _():
        for b in range(_NB):
            in_copy(b, b).start()

    for b in range(_NB):
        ch = i * _NB + b

        @pl.when(ch >= _NB)
        def _():
            out_copy(ch - _NB, b).wait()

        in_copy(ch, b).wait()
        h = jnp.tanh(
            jnp.dot(
                ibufs[b][...].astype(jnp.bfloat16),
                w1_ref[...],
                preferred_element_type=jnp.float32,
            )
            + b1_ref[...]
        )
        obufs[b][...] = jnp.dot(
            h.astype(jnp.bfloat16), w23_ref[...], preferred_element_type=jnp.float32
        )
        out_copy(ch, b).start()

        @pl.when(ch + _NB < _NCHUNK)
        def _():
            in_copy(ch + _NB, b).start()

    @pl.when(i == _NSTEP - 1)
    def _():
        for b in range(_NB):
            out_copy((_NSTEP - 1) * _NB + b, b).wait()


def _precompute_g(table, w1p, b1p, w23p):
    return pl.pallas_call(
        _g_manual_body,
        grid=(_NSTEP,),
        in_specs=[
            pl.BlockSpec(memory_space=pltpu.MemorySpace.HBM),
            pl.BlockSpec((_EMBED, _HPAD), lambda i: (0, 0)),
            pl.BlockSpec((1, _HPAD), lambda i: (0, 0)),
            pl.BlockSpec((_HPAD, _GW), lambda i: (0, 0)),
        ],
        out_specs=pl.BlockSpec(memory_space=pltpu.MemorySpace.HBM),
        out_shape=jax.ShapeDtypeStruct((_VOCAB, _GW), jnp.float32),
        scratch_shapes=(
            [pltpu.VMEM((_CH, _EMBED), jnp.float32) for _ in range(_NB)]
            + [pltpu.VMEM((_CH, _GW), jnp.float32) for _ in range(_NB)]
            + [pltpu.SemaphoreType.DMA for _ in range(2 * _NB)]
        ),
        compiler_params=pltpu.CompilerParams(vmem_limit_bytes=100 * 1024 * 1024),
    )(table, w1p, b1p, w23p)


def _reduce_elem(rows_ref, acc_ref, e, off=0):
    def red(j, accs):
        a0, a1, a2, a3 = accs
        r = off + j * 8
        a0 = a0 + rows_ref[r] + rows_ref[r + 4]
        a1 = a1 + rows_ref[r + 1] + rows_ref[r + 5]
        a2 = a2 + rows_ref[r + 2] + rows_ref[r + 6]
        a3 = a3 + rows_ref[r + 3] + rows_ref[r + 7]
        return (a0, a1, a2, a3)

    z = jnp.zeros((_GW,), jnp.float32)
    a0, a1, a2, a3 = lax.fori_loop(0, _SEQ // 8, red, (z, z, z, z))
    acc_ref[e] = (a0 + a1) + (a2 + a3)


_CHK = 4
_CROWS = _CHK * _SEQ
_NCHK = _EPW // _CHK


def _sc_body(idx_hbm, g_hbm, out_hbm, idx_v, *rest):
    rows = rest[:_NBUF]
    acc_v = rest[_NBUF]
    sems = rest[_NBUF + 1 : _NBUF + 1 + _NBUF]
    c = lax.axis_index("c")
    s = lax.axis_index("s")
    wid = s * _NC + c
    base = wid * _EPW

    pltpu.sync_copy(idx_hbm.at[pl.ds(base * _SEQ, _EPW * _SEQ)], idx_v)

    def fire(ch, buf, sem):
        pltpu.make_async_copy(
            g_hbm.at[idx_v.at[pl.ds(ch * _CROWS, _CROWS)]], buf, sem
        ).start()

    def drain(ch, buf, sem):
        pltpu.make_async_copy(
            g_hbm.at[idx_v.at[pl.ds(ch * _CROWS, _CROWS)]], buf, sem
        ).wait()

    for b in range(_NBUF):
        fire(b, rows[b], sems[b])

    def body(i, carry):
        c0 = _NBUF * i
        for b in range(_NBUF):
            ch = c0 + b
            drain(ch, rows[b], sems[b])
            for k in range(_CHK):
                _reduce_elem(rows[b], acc_v, ch * _CHK + k, off=k * _SEQ)

            @pl.when(ch + _NBUF < _NCHK)
            def _():
                fire(ch + _NBUF, rows[b], sems[b])

        return carry

    lax.fori_loop(0, _NCHK // _NBUF, body, 0)
    pltpu.sync_copy(acc_v, out_hbm.at[pl.ds(base, _EPW)])


def _sc_gather_sum(idx1, g):
    mesh = plsc.VectorSubcoreMesh(core_axis_name="c", subcore_axis_name="s")
    return pl.kernel(
        _sc_body,
        out_type=jax.ShapeDtypeStruct((_BATCH, _GW), jnp.float32),
        mesh=mesh,
        scratch_types=(
            [pltpu.VMEM((_EPW * _SEQ,), jnp.int32)]
            + [pltpu.VMEM((_CROWS, _GW), jnp.float32) for _ in range(_NBUF)]
            + [pltpu.VMEM((_EPW, _GW), jnp.float32)]
            + [pltpu.SemaphoreType.DMA for _ in range(_NBUF)]
        ),
        compiler_params=pltpu.CompilerParams(use_tc_tiling_on_sc=False),
    )(idx1, g)


def kernel(inputs, table, W1, b1, W2, b2, W3, b3):
    idx1 = inputs.astype(jnp.int32).reshape(_BATCH * _SEQ)
    w23 = W3 @ W2
    w1p = (
        jnp.zeros((_EMBED, _HPAD), jnp.float32)
        .at[:, :100]
        .set(W1.T)
        .astype(jnp.bfloat16)
    )
    b1p = jnp.zeros((1, _HPAD), jnp.float32).at[0, :100].set(b1)
    w23p = (
        jnp.zeros((_HPAD, _GW), jnp.float32)
        .at[:100, :5]
        .set(w23.T)
        .astype(jnp.bfloat16)
    )
    g = _precompute_g(table, w1p, b1p, w23p)
    s16 = _sc_gather_sum(idx1, g)
    const = _SEQ * (b2 @ W3.T + b3)
    return s16[:, :5] + const


# speedup vs baseline: 1.1580x; 1.0053x over previous
import functools

import jax
import jax.numpy as jnp
from jax import lax
from jax.experimental import pallas as pl
from jax.experimental.pallas import tpu as pltpu
from jax.experimental.pallas import tpu_sc as plsc

_VOCAB = 1_000_000
_EMBED = 64
_BATCH = 4096
_SEQ = 200
_HPAD = 128
_GW = 16

_ROW_BLK = 20000
_NC = 2
_NS = 16
_NW = _NC * _NS
_EPW = _BATCH // _NW
_HSEQ = _SEQ // 2
_NBUF = 8


def _g_body(tbl_ref, w1_ref, b1_ref, w23_ref, out_ref):
    h = jnp.tanh(
        jnp.dot(
            tbl_ref[...].astype(jnp.bfloat16),
            w1_ref[...],
            preferred_element_type=jnp.float32,
        )
        + b1_ref[...]
    )
    out_ref[...] = jnp.dot(
        h.astype(jnp.bfloat16), w23_ref[...], preferred_element_type=jnp.float32
    )


_CH = 10000
_NCHUNK = _VOCAB // _CH
_NB = 5
_NSTEP = _NCHUNK // _NB


def _g_manual_body(tbl_hbm, w1_ref, b1_ref, w23_ref, out_hbm, *rest):
    ibufs = rest[:_NB]
    obufs = rest[_NB : 2 * _NB]
    isems = rest[2 * _NB : 3 * _NB]
    osems = rest[3 * _NB : 4 * _NB]
    i = pl.program_id(0)

    def in_copy(ch, b):
        return pltpu.make_async_copy(
            tbl_hbm.at[pl.ds(ch * _CH, _CH), :], ibufs[b], isems[b]
        )

    def out_copy(ch, b):
        return pltpu.make_async_copy(
            obufs[b], out_hbm.at[pl.ds(ch * _CH, _CH), :], osems[b]
        )

    @pl.when(i == 0)
    def _():
        for b in range(_NB):
            in_copy(b, b).start()

    for b in range(_NB):
        ch = i * _NB + b

        @pl.when(ch >= _NB)
        def _():
            out_copy(ch - _NB, b).wait()

        in_copy(ch, b).wait()
        h = jnp.tanh(
            jnp.dot(
                ibufs[b][...],
                w1_ref[...],
                preferred_element_type=jnp.float32,
            )
            + b1_ref[...]
        )
        obufs[b][...] = jnp.dot(
            h.astype(jnp.bfloat16), w23_ref[...], preferred_element_type=jnp.float32
        )
        out_copy(ch, b).start()

        @pl.when(ch + _NB < _NCHUNK)
        def _():
            in_copy(ch + _NB, b).start()

    @pl.when(i == _NSTEP - 1)
    def _():
        for b in range(_NB):
            out_copy((_NSTEP - 1) * _NB + b, b).wait()


def _precompute_g(table, w1p, b1p, w23p):
    return pl.pallas_call(
        _g_manual_body,
        grid=(_NSTEP,),
        in_specs=[
            pl.BlockSpec(memory_space=pltpu.MemorySpace.HBM),
            pl.BlockSpec((_EMBED, _HPAD), lambda i: (0, 0)),
            pl.BlockSpec((1, _HPAD), lambda i: (0, 0)),
            pl.BlockSpec((_HPAD, _GW), lambda i: (0, 0)),
        ],
        out_specs=pl.BlockSpec(memory_space=pltpu.MemorySpace.HBM),
        out_shape=jax.ShapeDtypeStruct((_VOCAB, _GW), jnp.float32),
        scratch_shapes=(
            [pltpu.VMEM((_CH, _EMBED), jnp.bfloat16) for _ in range(_NB)]
            + [pltpu.VMEM((_CH, _GW), jnp.float32) for _ in range(_NB)]
            + [pltpu.SemaphoreType.DMA for _ in range(2 * _NB)]
        ),
        compiler_params=pltpu.CompilerParams(vmem_limit_bytes=100 * 1024 * 1024),
    )(table, w1p, b1p, w23p)


def _reduce_elem(rows_ref, acc_ref, e, off=0):
    def red(j, accs):
        a0, a1, a2, a3 = accs
        r = off + j * 8
        a0 = a0 + rows_ref[r] + rows_ref[r + 4]
        a1 = a1 + rows_ref[r + 1] + rows_ref[r + 5]
        a2 = a2 + rows_ref[r + 2] + rows_ref[r + 6]
        a3 = a3 + rows_ref[r + 3] + rows_ref[r + 7]
        return (a0, a1, a2, a3)

    z = jnp.zeros((_GW,), jnp.float32)
    a0, a1, a2, a3 = lax.fori_loop(0, _SEQ // 8, red, (z, z, z, z))
    acc_ref[e] = (a0 + a1) + (a2 + a3)


_CHK = 4
_CROWS = _CHK * _SEQ
_NCHK = _EPW // _CHK


def _sc_body(idx_hbm, g_hbm, out_hbm, idx_v, *rest):
    rows = rest[:_NBUF]
    acc_v = rest[_NBUF]
    sems = rest[_NBUF + 1 : _NBUF + 1 + _NBUF]
    c = lax.axis_index("c")
    s = lax.axis_index("s")
    wid = s * _NC + c
    base = wid * _EPW

    pltpu.sync_copy(idx_hbm.at[pl.ds(base * _SEQ, _EPW * _SEQ)], idx_v)

    def fire(ch, buf, sem):
        pltpu.make_async_copy(
            g_hbm.at[idx_v.at[pl.ds(ch * _CROWS, _CROWS)]], buf, sem
        ).start()

    def drain(ch, buf, sem):
        pltpu.make_async_copy(
            g_hbm.at[idx_v.at[pl.ds(ch * _CROWS, _CROWS)]], buf, sem
        ).wait()

    for b in range(_NBUF):
        fire(b, rows[b], sems[b])

    def body(i, carry):
        c0 = _NBUF * i
        for b in range(_NBUF):
            ch = c0 + b
            drain(ch, rows[b], sems[b])
            for k in range(_CHK):
                _reduce_elem(rows[b], acc_v, ch * _CHK + k, off=k * _SEQ)

            @pl.when(ch + _NBUF < _NCHK)
            def _():
                fire(ch + _NBUF, rows[b], sems[b])

        return carry

    lax.fori_loop(0, _NCHK // _NBUF, body, 0)
    pltpu.sync_copy(acc_v, out_hbm.at[pl.ds(base, _EPW)])


def _sc_gather_sum(idx1, g):
    mesh = plsc.VectorSubcoreMesh(core_axis_name="c", subcore_axis_name="s")
    return pl.kernel(
        _sc_body,
        out_type=jax.ShapeDtypeStruct((_BATCH, _GW), jnp.float32),
        mesh=mesh,
        scratch_types=(
            [pltpu.VMEM((_EPW * _SEQ,), jnp.int32)]
            + [pltpu.VMEM((_CROWS, _GW), jnp.float32) for _ in range(_NBUF)]
            + [pltpu.VMEM((_EPW, _GW), jnp.float32)]
            + [pltpu.SemaphoreType.DMA for _ in range(_NBUF)]
        ),
        compiler_params=pltpu.CompilerParams(use_tc_tiling_on_sc=False),
    )(idx1, g)


def kernel(inputs, table, W1, b1, W2, b2, W3, b3):
    idx1 = inputs.astype(jnp.int32).reshape(_BATCH * _SEQ)
    w23 = W3 @ W2
    w1p = (
        jnp.zeros((_EMBED, _HPAD), jnp.float32)
        .at[:, :100]
        .set(W1.T)
        .astype(jnp.bfloat16)
    )
    b1p = jnp.zeros((1, _HPAD), jnp.float32).at[0, :100].set(b1)
    w23p = (
        jnp.zeros((_HPAD, _GW), jnp.float32)
        .at[:100, :5]
        .set(w23.T)
        .astype(jnp.bfloat16)
    )
    g = _precompute_g(table.astype(jnp.bfloat16), w1p, b1p, w23p)
    s16 = _sc_gather_sum(idx1, g)
    const = _SEQ * (b2 @ W3.T + b3)
    return s16[:, :5] + const
